# baseline (speedup 1.0000x reference)
import functools

import jax
import jax.numpy as jnp
from jax import lax
from jax.experimental import pallas as pl
from jax.experimental.pallas import tpu as pltpu
from jax.experimental.pallas import tpu_sc as plsc

SIZE_ROWS = 16384
OBS_D = 1024
BLK = 2048
NBLK = SIZE_ROWS // BLK

NW = 32
EL_W = SIZE_ROWS // NW

_sc_mesh = plsc.VectorSubcoreMesh(core_axis_name="c", subcore_axis_name="s")


@functools.partial(
    pl.kernel,
    out_type=[
        jax.ShapeDtypeStruct((SIZE_ROWS,), jnp.float32),
        jax.ShapeDtypeStruct((SIZE_ROWS,), jnp.float32),
    ],
    mesh=_sc_mesh,
    scratch_types=[
        pltpu.VMEM((2, EL_W + 128), jnp.float32),
        pltpu.VMEM((2, EL_W), jnp.float32),
        pltpu.VMEM((2, 16), jnp.float32),
        pltpu.SemaphoreType.DMA,
        pltpu.SemaphoreType.DMA,
        pltpu.SemaphoreType.DMA,
    ],
)
def _sc_small_roll(act_in, rew_in, newvals, act_out, rew_out,
                   src_v, dst_v, nv_v, in_sem, nv_sem, out_sem):
    wid = lax.axis_index("s") * 2 + lax.axis_index("c")
    base = wid * EL_W
    ins = (act_in, rew_in)
    outs = (act_out, rew_out)

    pltpu.make_async_copy(newvals, nv_v, nv_sem).start()

    for a in range(2):
        @pl.when(wid == 0)
        def _():
            pltpu.make_async_copy(ins[a].at[pl.ds(0, EL_W)],
                                  src_v.at[a].at[pl.ds(128, EL_W)],
                                  in_sem).start()
            pltpu.make_async_copy(ins[a].at[pl.ds(0, 128)],
                                  src_v.at[a].at[pl.ds(0, 128)],
                                  in_sem).start()

        @pl.when(wid != 0)
        def _():
            pltpu.make_async_copy(ins[a].at[pl.ds(base - 128, EL_W + 128)],
                                  src_v.at[a], in_sem).start()
    for a in range(2):
        pltpu.make_async_copy(ins[a].at[pl.ds(0, EL_W + 128)], src_v.at[a],
                              in_sem).wait()
    pltpu.make_async_copy(newvals, nv_v, nv_sem).wait()

    lane = lax.iota(jnp.int32, 16)
    for a in range(2):
        for k in range(EL_W // 16):
            dst_v[a, pl.ds(16 * k, 16)] = src_v[a, pl.ds(16 * k + 127, 16)]

        @pl.when(wid == 0)
        def _():
            dst_v[a, pl.ds(0, 16)] = jnp.where(
                lane == 0, nv_v[a, :], src_v[a, pl.ds(127, 16)])

        pltpu.make_async_copy(dst_v.at[a], outs[a].at[pl.ds(base, EL_W)],
                              out_sem).start()
    for a in range(2):
        pltpu.make_async_copy(dst_v.at[a], outs[a].at[pl.ds(base, EL_W)],
                              out_sem).wait()


def _tc_body(cur_ref, prev_ref, obs_new_ref, obs_out_ref):
    i = pl.program_id(0)
    obs_out_ref[...] = pltpu.roll(cur_ref[...], 1, 0)
    row0 = jnp.where(i == 0, obs_new_ref[0, :], prev_ref[7, :])
    obs_out_ref[0:1, :] = row0[None, :]


def kernel(buffer_observations, buffer_actions, buffer_rewards,
           observation, action, reward):
    obs_new = observation.reshape(1, OBS_D)
    newvals = jnp.broadcast_to(jnp.stack([action, reward])[:, None], (2, 16))

    grid = (NBLK,)
    obs_out = pl.pallas_call(
        _tc_body,
        grid=grid,
        in_specs=[
            pl.BlockSpec((BLK, OBS_D), lambda i: (i, 0)),
            pl.BlockSpec((8, OBS_D), lambda i: (jnp.maximum(i * (BLK // 8) - 1, 0), 0)),
            pl.BlockSpec((1, OBS_D), lambda i: (0, 0)),
        ],
        out_specs=pl.BlockSpec((BLK, OBS_D), lambda i: (i, 0)),
        out_shape=jax.ShapeDtypeStruct((SIZE_ROWS, OBS_D), jnp.float32),
        compiler_params=pltpu.CompilerParams(
            dimension_semantics=("arbitrary",),
        ),
    )(buffer_observations, buffer_observations, obs_new)

    act_out, rew_out = _sc_small_roll(buffer_actions, buffer_rewards, newvals)

    return (obs_out, act_out, rew_out)


# speedup vs baseline: 1.3760x; 1.3760x over previous
import jax
import jax.numpy as jnp
from jax import lax
from jax.experimental import pallas as pl
from jax.experimental.pallas import tpu as pltpu

SIZE_ROWS = 16384
OBS_D = 1024
BLK = 2048
NBLK = SIZE_ROWS // BLK
VR, VC = 128, 128


def _shift_flat(x, newval):
    r, c = x.shape
    within = pltpu.roll(x, 1, 1)
    col = pltpu.roll(x[:, c - 1:c], 1, 0)
    ridx = lax.broadcasted_iota(jnp.int32, (r, c), 0)
    cidx = lax.broadcasted_iota(jnp.int32, (r, c), 1)
    out = jnp.where(cidx == 0, jnp.broadcast_to(col, (r, c)), within)
    return jnp.where((ridx == 0) & (cidx == 0), newval, out)


def _body(cur_ref, prev_ref, obs_new_ref, act_ref, rew_ref, scal_ref,
          obs_out_ref, act_out_ref, rew_out_ref):
    i = pl.program_id(0)
    obs_out_ref[...] = pltpu.roll(cur_ref[...], 1, 0)
    row0 = jnp.where(i == 0, obs_new_ref[0, :], prev_ref[7, :])
    obs_out_ref[0:1, :] = row0[None, :]

    @pl.when(i == 0)
    def _small():
        act_out_ref[...] = _shift_flat(act_ref[...], scal_ref[0, 0])
        rew_out_ref[...] = _shift_flat(rew_ref[...], scal_ref[0, 1])


def kernel(buffer_observations, buffer_actions, buffer_rewards,
           observation, action, reward):
    act2d = buffer_actions.reshape(VR, VC)
    rew2d = buffer_rewards.reshape(VR, VC)
    obs_new = observation.reshape(1, OBS_D)
    scal = jnp.stack([action, reward]).reshape(1, 2)

    grid = (NBLK,)
    obs_out, act_out, rew_out = pl.pallas_call(
        _body,
        grid=grid,
        in_specs=[
            pl.BlockSpec((BLK, OBS_D), lambda i: (i, 0)),
            pl.BlockSpec((8, OBS_D), lambda i: (jnp.maximum(i * (BLK // 8) - 1, 0), 0)),
            pl.BlockSpec((1, OBS_D), lambda i: (0, 0)),
            pl.BlockSpec((VR, VC), lambda i: (0, 0)),
            pl.BlockSpec((VR, VC), lambda i: (0, 0)),
            pl.BlockSpec(memory_space=pltpu.SMEM),
        ],
        out_specs=[
            pl.BlockSpec((BLK, OBS_D), lambda i: (i, 0)),
            pl.BlockSpec((VR, VC), lambda i: (0, 0)),
            pl.BlockSpec((VR, VC), lambda i: (0, 0)),
        ],
        out_shape=[
            jax.ShapeDtypeStruct((SIZE_ROWS, OBS_D), jnp.float32),
            jax.ShapeDtypeStruct((VR, VC), jnp.float32),
            jax.ShapeDtypeStruct((VR, VC), jnp.float32),
        ],
        compiler_params=pltpu.CompilerParams(
            dimension_semantics=("parallel",),
            vmem_limit_bytes=128 * 1024 * 1024,
        ),
    )(buffer_observations, buffer_observations, obs_new, act2d, rew2d, scal)

    return (obs_out, act_out.reshape(SIZE_ROWS), rew_out.reshape(SIZE_ROWS))
